# baseline (speedup 1.0000x reference)
import jax
import jax.numpy as jnp
from jax import lax
from jax.experimental import pallas as pl
from jax.experimental.pallas import tpu as pltpu
from jax.experimental.pallas import tpu_sc as plsc

N_NODES = 10000
N_EDGES = 320000
D_HID = 128
N_CLASSES = 64

NC = 2
NS = 16
NW = NC * NS
CK = 128
NCK = 80
CPW = NCK + 1
N_ACC = N_NODES + 8
GR = 80
NG = N_NODES // GR
GPT = (NG + NS - 1) // NS


def _make_agg(d):
  mesh = plsc.VectorSubcoreMesh(core_axis_name="c", subcore_axis_name="s",
                                num_cores=NC, num_subcores=NS)

  def body(h_hbm, src_hbm, dst_hbm, out_hbm,
           s0, s1, d0, d1, r0, r1, zbuf, acc_sh, g0, g1):
    sv = (s0, s1)
    dv = (d0, d1)
    rv = (r0, r1)
    gs = (g0, g1)
    cid = lax.axis_index("c")
    sid = lax.axis_index("s")
    wid = sid * NC + cid
    e_base = wid * CPW * CK

    zero16 = jnp.zeros((16,), jnp.float32)

    def zrow(r, carry):
      for j in range(d // 16):
        zbuf[r, pl.ds(j * 16, 16)] = zero16
      return carry

    lax.fori_loop(0, GR, zrow, 0)
    for it in range(GPT):
      g = sid + it * NS

      @pl.when(g < NG)
      def _():
        pltpu.sync_copy(zbuf, acc_sh.at[pl.ds(g * GR, GR)])

    @pl.when(sid == NS - 1)
    def _():
      pltpu.sync_copy(zbuf.at[pl.ds(0, N_ACC - N_NODES)],
                      acc_sh.at[pl.ds(N_NODES, N_ACC - N_NODES)])

    plsc.subcore_barrier()

    pltpu.sync_copy(src_hbm.at[pl.ds(e_base, CK)], sv[0])
    pltpu.sync_copy(dst_hbm.at[pl.ds(e_base, CK)], dv[0])
    pltpu.async_copy(h_hbm.at[sv[0]], rv[0], gs[0])

    def step(i, carry):
      for u in range(2):
        cur = i * 2 + u
        b = u
        b2 = 1 - u
        e1 = e_base + (cur + 1) * CK
        pltpu.sync_copy(src_hbm.at[pl.ds(e1, CK)], sv[b2])
        pltpu.sync_copy(dst_hbm.at[pl.ds(e1, CK)], dv[b2])
        pltpu.make_async_copy(h_hbm.at[sv[b]], rv[b], gs[b]).wait()
        pltpu.async_copy(h_hbm.at[sv[b2]], rv[b2], gs[b2])
        pltpu.sync_copy(rv[b], acc_sh.at[dv[b]], add=True)
      return carry

    lax.fori_loop(0, NCK // 2, step, 0)
    pltpu.make_async_copy(h_hbm.at[sv[0]], rv[0], gs[0]).wait()

    plsc.subcore_barrier()
    for it in range(GPT):
      g = sid + it * NS

      @pl.when(g < NG)
      def _():
        pltpu.sync_copy(acc_sh.at[pl.ds(g * GR, GR)], zbuf)
        pltpu.sync_copy(zbuf, out_hbm.at[cid, pl.ds(g * GR, GR)])

  return pl.kernel(
      body,
      out_type=jax.ShapeDtypeStruct((NC, N_NODES, d), jnp.float32),
      mesh=mesh,
      compiler_params=pltpu.CompilerParams(use_tc_tiling_on_sc=(d % 128 == 0)),
      scratch_types=[
          pltpu.VMEM((CK,), jnp.int32),
          pltpu.VMEM((CK,), jnp.int32),
          pltpu.VMEM((CK,), jnp.int32),
          pltpu.VMEM((CK,), jnp.int32),
          pltpu.VMEM((CK, d), jnp.float32),
          pltpu.VMEM((CK, d), jnp.float32),
          pltpu.VMEM((GR, d), jnp.float32),
          pltpu.VMEM_SHARED((N_ACC, d), jnp.float32),
          pltpu.SemaphoreType.DMA,
          pltpu.SemaphoreType.DMA,
      ],
  )


_AGG_HID = _make_agg(D_HID)
_AGG_CLS = _make_agg(N_CLASSES)


def _mm_bias(x_ref, w_ref, b_ref, o_ref):
  o_ref[...] = jnp.dot(x_ref[...], w_ref[...],
                       preferred_element_type=jnp.float32) + b_ref[...]


def _combine_mm_bias(p_ref, w_ref, b_ref, o_ref):
  x = jnp.maximum(p_ref[0] + p_ref[1], 0.0)
  o_ref[...] = jnp.dot(x, w_ref[...],
                       preferred_element_type=jnp.float32) + b_ref[...]


def _combine_relu(p_ref, o_ref):
  o_ref[...] = jnp.maximum(p_ref[0] + p_ref[1], 0.0)


def kernel(node_features, edge_index, W1, b1, W2, b2):
  x = node_features.astype(jnp.float32)
  ei = edge_index.astype(jnp.int32)
  src, dst = ei[0], ei[1]

  pad = NW * NCK * CK - N_EDGES
  src_p = jnp.concatenate([src, jnp.zeros((pad,), jnp.int32)])
  src_p = src_p.reshape(NW, NCK, CK)
  src_p = jnp.concatenate(
      [src_p, jnp.zeros((NW, CPW - NCK, CK), jnp.int32)], axis=1)
  src_p = src_p.reshape(NW * CPW * CK)
  dst_p = jnp.concatenate([dst, jnp.full((pad,), N_NODES, jnp.int32)])
  dst_p = dst_p.reshape(NW, NCK, CK)
  dst_p = jnp.concatenate(
      [dst_p, jnp.full((NW, CPW - NCK, CK), N_NODES, jnp.int32)], axis=1)
  dst_p = dst_p.reshape(NW * CPW * CK)

  h1 = pl.pallas_call(
      _mm_bias,
      out_shape=jax.ShapeDtypeStruct((N_NODES, D_HID), jnp.float32),
  )(x, W1, b1.reshape(1, D_HID))

  p1 = _AGG_HID(h1, src_p, dst_p)

  h2 = pl.pallas_call(
      _combine_mm_bias,
      out_shape=jax.ShapeDtypeStruct((N_NODES, N_CLASSES), jnp.float32),
  )(p1, W2, b2.reshape(1, N_CLASSES))

  p2 = _AGG_CLS(h2, src_p, dst_p)

  out = pl.pallas_call(
      _combine_relu,
      out_shape=jax.ShapeDtypeStruct((N_NODES, N_CLASSES), jnp.float32),
  )(p2)
  return out


# speedup vs baseline: 2.9496x; 2.9496x over previous
import jax
import jax.numpy as jnp
from jax import lax
from jax.experimental import pallas as pl
from jax.experimental.pallas import tpu as pltpu
from jax.experimental.pallas import tpu_sc as plsc

N_NODES = 10000
N_EDGES = 320000
D_HID = 128
N_CLASSES = 64

NC = 2
NS = 16
NW = NC * NS
EPW = N_EDGES // NW
CK = 128
NFULL = EPW // CK
TAIL = EPW - NFULL * CK
GR = 80
NG = N_NODES // GR
GPT = (NG + NS - 1) // NS


def _make_agg(d):
  mesh = plsc.VectorSubcoreMesh(core_axis_name="c", subcore_axis_name="s",
                                num_cores=NC, num_subcores=NS)

  def body(h_hbm, src_hbm, dst_hbm, out_hbm,
           s0, s1, d0, d1, r0, r1, src_t, dst_t, rows_t, zbuf, acc_sh,
           g0, g1):
    sv = (s0, s1)
    dv = (d0, d1)
    rv = (r0, r1)
    gs = (g0, g1)
    cid = lax.axis_index("c")
    sid = lax.axis_index("s")
    wid = sid * NC + cid
    e_base = wid * EPW

    zero16 = jnp.zeros((16,), jnp.float32)

    def zrow(r, carry):
      for j in range(d // 16):
        zbuf[r, pl.ds(j * 16, 16)] = zero16
      return carry

    lax.fori_loop(0, GR, zrow, 0)
    for it in range(GPT):
      g = sid + it * NS

      @pl.when(g < NG)
      def _():
        pltpu.sync_copy(zbuf, acc_sh.at[pl.ds(g * GR, GR)])

    plsc.subcore_barrier()

    pltpu.sync_copy(src_hbm.at[pl.ds(e_base, CK)], sv[0])
    pltpu.sync_copy(dst_hbm.at[pl.ds(e_base, CK)], dv[0])
    pltpu.async_copy(h_hbm.at[sv[0]], rv[0], gs[0])

    def step(i, carry):
      for u in range(2):
        cur = i * 2 + u
        b = u
        b2 = 1 - u
        e1 = e_base + (cur + 1) * CK
        pltpu.sync_copy(src_hbm.at[pl.ds(e1, CK)], sv[b2])
        pltpu.sync_copy(dst_hbm.at[pl.ds(e1, CK)], dv[b2])
        pltpu.make_async_copy(h_hbm.at[sv[b]], rv[b], gs[b]).wait()
        pltpu.async_copy(h_hbm.at[sv[b2]], rv[b2], gs[b2])
        pltpu.sync_copy(rv[b], acc_sh.at[dv[b]], add=True)
      return carry

    lax.fori_loop(0, NFULL // 2, step, 0)
    pltpu.make_async_copy(h_hbm.at[sv[0]], rv[0], gs[0]).wait()

    e0 = e_base + NFULL * CK
    pltpu.sync_copy(src_hbm.at[pl.ds(e0, TAIL)], src_t)
    pltpu.sync_copy(dst_hbm.at[pl.ds(e0, TAIL)], dst_t)
    pltpu.async_copy(h_hbm.at[src_t], rows_t, gs[0]).wait()
    pltpu.sync_copy(rows_t, acc_sh.at[dst_t], add=True)

    plsc.subcore_barrier()
    for it in range(GPT):
      g = sid + it * NS

      @pl.when(g < NG)
      def _():
        pltpu.sync_copy(acc_sh.at[pl.ds(g * GR, GR)], zbuf)
        pltpu.sync_copy(zbuf, out_hbm.at[cid, pl.ds(g * GR, GR)])

  return pl.kernel(
      body,
      out_type=jax.ShapeDtypeStruct((NC, N_NODES, d), jnp.float32),
      mesh=mesh,
      compiler_params=pltpu.CompilerParams(use_tc_tiling_on_sc=(d % 128 == 0)),
      scratch_types=[
          pltpu.VMEM((CK,), jnp.int32),
          pltpu.VMEM((CK,), jnp.int32),
          pltpu.VMEM((CK,), jnp.int32),
          pltpu.VMEM((CK,), jnp.int32),
          pltpu.VMEM((CK, d), jnp.float32),
          pltpu.VMEM((CK, d), jnp.float32),
          pltpu.VMEM((TAIL,), jnp.int32),
          pltpu.VMEM((TAIL,), jnp.int32),
          pltpu.VMEM((TAIL, d), jnp.float32),
          pltpu.VMEM((GR, d), jnp.float32),
          pltpu.VMEM_SHARED((N_NODES, d), jnp.float32),
          pltpu.SemaphoreType.DMA,
          pltpu.SemaphoreType.DMA,
      ],
  )


_AGG_HID = _make_agg(D_HID)
_AGG_CLS = _make_agg(N_CLASSES)


def _mm_bias(x_ref, w_ref, b_ref, o_ref):
  o_ref[...] = jnp.dot(x_ref[...], w_ref[...],
                       preferred_element_type=jnp.float32) + b_ref[...]


def _combine_mm_bias(p_ref, w_ref, b_ref, o_ref):
  x = jnp.maximum(p_ref[0] + p_ref[1], 0.0)
  o_ref[...] = jnp.dot(x, w_ref[...],
                       preferred_element_type=jnp.float32) + b_ref[...]


def _combine_relu(p_ref, o_ref):
  o_ref[...] = jnp.maximum(p_ref[0] + p_ref[1], 0.0)


def kernel(node_features, edge_index, W1, b1, W2, b2):
  x = node_features.astype(jnp.float32)
  ei = edge_index.astype(jnp.int32)
  src, dst = ei[0], ei[1]

  slack = jnp.zeros((CK,), jnp.int32)
  src_p = jnp.concatenate([src, slack])
  dst_p = jnp.concatenate([dst, slack])

  h1 = pl.pallas_call(
      _mm_bias,
      out_shape=jax.ShapeDtypeStruct((N_NODES, D_HID), jnp.float32),
  )(x, W1, b1.reshape(1, D_HID))

  p1 = _AGG_HID(h1, src_p, dst_p)

  h2 = pl.pallas_call(
      _combine_mm_bias,
      out_shape=jax.ShapeDtypeStruct((N_NODES, N_CLASSES), jnp.float32),
  )(p1, W2, b2.reshape(1, N_CLASSES))

  p2 = _AGG_CLS(h2, src_p, dst_p)

  out = pl.pallas_call(
      _combine_relu,
      out_shape=jax.ShapeDtypeStruct((N_NODES, N_CLASSES), jnp.float32),
  )(p2)
  return out
